# baseline (speedup 1.0000x reference)
import functools

import jax
import jax.numpy as jnp
from jax import lax
from jax.experimental import pallas as pl
from jax.experimental.pallas import tpu as pltpu, tpu_sc as plsc

_INFO = plsc.get_sparse_core_info()
_NC, _NS = _INFO.num_cores, _INFO.num_subcores
_NW = _NC * _NS

_B = 327680
_PER_W = _B // _NW
_UNITS = _PER_W // 128


_NG = 7812
_NG_PER_W = _NG // _NW


@jax.jit
def _retile(et, tail):
    mesh = plsc.VectorSubcoreMesh(core_axis_name="c", subcore_axis_name="s")

    @functools.partial(
        pl.kernel,
        mesh=mesh,
        out_type=jax.ShapeDtypeStruct((250000, 128), jnp.float32),
        scratch_types=[
            pltpu.VMEM((32, 128), jnp.float32),
            pltpu.VMEM((32, 128), jnp.float32),
        ],
        compiler_params=pltpu.CompilerParams(needs_layout_passes=False),
    )
    def body(et_hbm, tail_hbm, sc_hbm, in_v, out_v):
        wid = lax.axis_index("s") * _NC + lax.axis_index("c")
        lane = lax.iota(jnp.int32, 16)

        def shuffle():
            for qq in range(32):
                for lg in range(8):
                    row_idx = ((lg & 1) << 4) + lane
                    colv = lane * 0 + (4 * qq + (lg >> 1))
                    out_v[qq, pl.ds(lg * 16, 16)] = plsc.load_gather(
                        in_v, [row_idx, colv])

        gbase = wid * _NG_PER_W + jnp.minimum(wid, _NG - _NG_PER_W * _NW)
        gcnt = _NG_PER_W + (wid < _NG - _NG_PER_W * _NW).astype(jnp.int32)

        def group(t, carry):
            g = gbase + t
            c0 = pl.multiple_of(g * 128, 128)
            pltpu.sync_copy(et_hbm.at[:, pl.ds(c0, 128)], in_v)
            shuffle()
            pltpu.sync_copy(out_v,
                            sc_hbm.at[pl.ds(pl.multiple_of(g * 32, 32), 32), :])
            return carry

        lax.fori_loop(0, gcnt, group, 0)

        @pl.when(wid == 0)
        def _():
            pltpu.sync_copy(tail_hbm, in_v.at[pl.ds(0, 16), :])
            pltpu.sync_copy(in_v.at[pl.ds(0, 16), :],
                            sc_hbm.at[pl.ds(249984, 16), :])

    return body(et, tail)


@jax.jit
def _embed_lookup(xflat, table):
    mesh = plsc.VectorSubcoreMesh(core_axis_name="c", subcore_axis_name="s")

    @functools.partial(
        pl.kernel,
        mesh=mesh,
        out_type=jax.ShapeDtypeStruct((20, 32, 16384), jnp.float32),
        scratch_types=[
            pltpu.VMEM((_PER_W,), jnp.int32),
            pltpu.VMEM((_PER_W,), jnp.int32),
            pltpu.VMEM((128, 128), jnp.float32),
            pltpu.VMEM((1, 32, 128), jnp.float32),
            pltpu.SemaphoreType.DMA,
        ],
        compiler_params=pltpu.CompilerParams(needs_layout_passes=False),
    )
    def body(table_hbm, xf_hbm, out_hbm, idx_v, q_v, rows_v, ot_v, sem):
        wid = lax.axis_index("s") * _NC + lax.axis_index("c")
        lo = wid * _PER_W
        pltpu.sync_copy(xf_hbm.at[pl.ds(lo, _PER_W)], idx_v)
        for v in range(_PER_W // 16):
            q_v[pl.ds(v * 16, 16)] = idx_v[pl.ds(v * 16, 16)] >> 2
        lane = lax.iota(jnp.int32, 16)

        def unit(t, carry):
            u = wid * _UNITS + t
            j = u >> 7
            i0 = pl.multiple_of((u & 127) << 7, 128)
            pltpu.async_copy(
                table_hbm.at[q_v.at[pl.ds(t * 128, 128)]], rows_v, sem
            ).wait()
            for lg in range(8):
                r16 = idx_v[pl.ds(t * 128 + lg * 16, 16)]
                row_idx = lane + lg * 16
                col0 = (r16 & 3) << 5
                for e in range(32):
                    ot_v[0, e, pl.ds(lg * 16, 16)] = plsc.load_gather(
                        rows_v, [row_idx, col0 + e])
            pltpu.sync_copy(ot_v, out_hbm.at[pl.ds(j, 1), :, pl.ds(i0, 128)])
            return carry

        lax.fori_loop(0, _UNITS, unit, 0)

    return body(table, xflat)


def kernel(x, embedding):
    if x.dtype != jnp.int32:
        x = jnp.round(x).astype(jnp.int32)
    table = _retile(embedding.T, embedding[999936:].reshape(16, 128))
    out_t = _embed_lookup(x.T.reshape(-1), table)
    return jnp.transpose(out_t, (2, 0, 1))


# speedup vs baseline: 1.2201x; 1.2201x over previous
import functools

import jax
import jax.numpy as jnp
from jax import lax
from jax.experimental import pallas as pl
from jax.experimental.pallas import tpu as pltpu, tpu_sc as plsc

_INFO = plsc.get_sparse_core_info()
_NC, _NS = _INFO.num_cores, _INFO.num_subcores
_NW = _NC * _NS

_B = 327680
_PER_W = _B // _NW
_UNITS = _PER_W // 128


_GL = 128
_NG = 999936 // _GL
_NG_PER_W = _NG // _NW


@jax.jit
def _retile(et, tail):
    mesh = plsc.VectorSubcoreMesh(core_axis_name="c", subcore_axis_name="s")

    @functools.partial(
        pl.kernel,
        mesh=mesh,
        out_type=jax.ShapeDtypeStruct((250000, 128), jnp.float32),
        scratch_types=[
            pltpu.VMEM((32, _GL), jnp.float32),
            pltpu.VMEM((32, _GL), jnp.float32),
            pltpu.VMEM((_GL // 4, 128), jnp.float32),
            pltpu.VMEM((_GL // 4, 128), jnp.float32),
            pltpu.SemaphoreType.DMA,
            pltpu.SemaphoreType.DMA,
            pltpu.SemaphoreType.DMA,
            pltpu.SemaphoreType.DMA,
        ],
        compiler_params=pltpu.CompilerParams(needs_layout_passes=False),
    )
    def body(et_hbm, tail_hbm, sc_hbm, in_v0, in_v1, out_v0, out_v1,
             isem0, isem1, osem0, osem1):
        wid = lax.axis_index("s") * _NC + lax.axis_index("c")
        lane = lax.iota(jnp.int32, 16)
        ins = (in_v0, in_v1)
        outs = (out_v0, out_v1)
        isems = (isem0, isem1)
        osems = (osem0, osem1)

        nx = _NG - _NG_PER_W * _NW
        gbase = wid * _NG_PER_W + jnp.minimum(wid, nx)
        gcnt = _NG_PER_W + (wid < nx).astype(jnp.int32)

        def start_in(t, ib):
            c0 = pl.multiple_of((gbase + t) * _GL, 128)
            pltpu.async_copy(et_hbm.at[:, pl.ds(c0, _GL)], ins[ib], isems[ib])

        def wait_in(ib):
            pltpu.make_async_copy(
                et_hbm.at[:, pl.ds(0, _GL)], ins[ib], isems[ib]).wait()

        def start_out(t, ib):
            r0 = pl.multiple_of((gbase + t) * (_GL // 4), 8)
            pltpu.async_copy(
                outs[ib], sc_hbm.at[pl.ds(r0, _GL // 4), :], osems[ib])

        def wait_out(ib):
            pltpu.make_async_copy(
                outs[ib], sc_hbm.at[pl.ds(0, _GL // 4), :], osems[ib]).wait()

        def shuffle(ib):
            for qq in range(_GL // 4):
                for lg in range(8):
                    row_idx = ((lg & 1) << 4) + lane
                    colv = lane * 0 + (4 * qq + (lg >> 1))
                    outs[ib][qq, pl.ds(lg * 16, 16)] = plsc.load_gather(
                        ins[ib], [row_idx, colv])

        def phase(t, ib):
            @pl.when(t < gcnt)
            def _():
                wait_in(ib)

                @pl.when(t + 1 < gcnt)
                def _():
                    start_in(t + 1, 1 - ib)

                @pl.when(t >= 2)
                def _():
                    wait_out(ib)

                shuffle(ib)
                start_out(t, ib)

        start_in(0, 0)

        def pair(s, carry):
            phase(2 * s, 0)
            phase(2 * s + 1, 1)
            return carry

        lax.fori_loop(0, (_NG_PER_W + 2) // 2 + 1, pair, 0)
        wait_out(0)
        wait_out(1)

        @pl.when(wid == 0)
        def _():
            pltpu.sync_copy(tail_hbm, in_v0.at[pl.ds(0, 16), pl.ds(0, 128)])
            pltpu.sync_copy(in_v0.at[pl.ds(0, 16), pl.ds(0, 128)],
                            sc_hbm.at[pl.ds(249984, 16), :])

    return body(et, tail)


@jax.jit
def _embed_lookup(xflat, table):
    mesh = plsc.VectorSubcoreMesh(core_axis_name="c", subcore_axis_name="s")

    @functools.partial(
        pl.kernel,
        mesh=mesh,
        out_type=jax.ShapeDtypeStruct((20, 32, 16384), jnp.float32),
        scratch_types=[
            pltpu.VMEM((_PER_W,), jnp.int32),
            pltpu.VMEM((_PER_W,), jnp.int32),
            pltpu.VMEM((128, 128), jnp.float32),
            pltpu.VMEM((1, 32, 128), jnp.float32),
            pltpu.SemaphoreType.DMA,
        ],
        compiler_params=pltpu.CompilerParams(needs_layout_passes=False),
    )
    def body(table_hbm, xf_hbm, out_hbm, idx_v, q_v, rows_v, ot_v, sem):
        wid = lax.axis_index("s") * _NC + lax.axis_index("c")
        lo = wid * _PER_W
        pltpu.sync_copy(xf_hbm.at[pl.ds(lo, _PER_W)], idx_v)
        for v in range(_PER_W // 16):
            q_v[pl.ds(v * 16, 16)] = idx_v[pl.ds(v * 16, 16)] >> 2
        lane = lax.iota(jnp.int32, 16)

        def unit(t, carry):
            u = wid * _UNITS + t
            j = u >> 7
            i0 = pl.multiple_of((u & 127) << 7, 128)
            pltpu.async_copy(
                table_hbm.at[q_v.at[pl.ds(t * 128, 128)]], rows_v, sem
            ).wait()
            for lg in range(8):
                r16 = idx_v[pl.ds(t * 128 + lg * 16, 16)]
                row_idx = lane + lg * 16
                col0 = (r16 & 3) << 5
                for e in range(32):
                    ot_v[0, e, pl.ds(lg * 16, 16)] = plsc.load_gather(
                        rows_v, [row_idx, col0 + e])
            pltpu.sync_copy(ot_v, out_hbm.at[pl.ds(j, 1), :, pl.ds(i0, 128)])
            return carry

        lax.fori_loop(0, _UNITS, unit, 0)

    return body(table, xflat)


def kernel(x, embedding):
    if x.dtype != jnp.int32:
        x = jnp.round(x).astype(jnp.int32)
    table = _retile(embedding.T, embedding[999936:].reshape(16, 128))
    out_t = _embed_lookup(x.T.reshape(-1), table)
    return jnp.transpose(out_t, (2, 0, 1))


# speedup vs baseline: 1.5984x; 1.3101x over previous
import functools

import jax
import jax.numpy as jnp
from jax import lax
from jax.experimental import pallas as pl
from jax.experimental.pallas import tpu as pltpu, tpu_sc as plsc

_INFO = plsc.get_sparse_core_info()
_NC, _NS = _INFO.num_cores, _INFO.num_subcores
_NW = _NC * _NS

_B = 327680
_PER_W = _B // _NW
_UNITS = _PER_W // 128


_GL = 128
_NG = 999936 // _GL
_NG_PER_W = _NG // _NW


@jax.jit
def _retile(et, tail):
    mesh = plsc.VectorSubcoreMesh(core_axis_name="c", subcore_axis_name="s")

    @functools.partial(
        pl.kernel,
        mesh=mesh,
        out_type=jax.ShapeDtypeStruct((250000, 128), jnp.float32),
        scratch_types=[
            pltpu.VMEM((32, _GL), jnp.float32),
            pltpu.VMEM((32, _GL), jnp.float32),
            pltpu.VMEM((_GL // 4, 128), jnp.float32),
            pltpu.VMEM((_GL // 4, 128), jnp.float32),
            pltpu.SemaphoreType.DMA,
            pltpu.SemaphoreType.DMA,
            pltpu.SemaphoreType.DMA,
            pltpu.SemaphoreType.DMA,
        ],
        compiler_params=pltpu.CompilerParams(needs_layout_passes=False),
    )
    def body(et_hbm, tail_hbm, sc_hbm, in_v0, in_v1, out_v0, out_v1,
             isem0, isem1, osem0, osem1):
        wid = lax.axis_index("s") * _NC + lax.axis_index("c")
        lane = lax.iota(jnp.int32, 16)
        ins = (in_v0, in_v1)
        outs = (out_v0, out_v1)
        isems = (isem0, isem1)
        osems = (osem0, osem1)

        nx = _NG - _NG_PER_W * _NW
        gbase = wid * _NG_PER_W + jnp.minimum(wid, nx)
        gcnt = _NG_PER_W + (wid < nx).astype(jnp.int32)

        def start_in(t, ib):
            c0 = pl.multiple_of((gbase + t) * _GL, 128)
            pltpu.async_copy(et_hbm.at[:, pl.ds(c0, _GL)], ins[ib], isems[ib])

        def wait_in(ib):
            pltpu.make_async_copy(
                et_hbm.at[:, pl.ds(0, _GL)], ins[ib], isems[ib]).wait()

        def start_out(t, ib):
            r0 = pl.multiple_of((gbase + t) * (_GL // 4), 8)
            pltpu.async_copy(
                outs[ib], sc_hbm.at[pl.ds(r0, _GL // 4), :], osems[ib])

        def wait_out(ib):
            pltpu.make_async_copy(
                outs[ib], sc_hbm.at[pl.ds(0, _GL // 4), :], osems[ib]).wait()

        def shuffle(ib):
            for qq in range(_GL // 4):
                for lg in range(8):
                    row_idx = ((lg & 1) << 4) + lane
                    colv = lane * 0 + (4 * qq + (lg >> 1))
                    outs[ib][qq, pl.ds(lg * 16, 16)] = plsc.load_gather(
                        ins[ib], [row_idx, colv])

        def phase(t, ib):
            @pl.when(t < gcnt)
            def _():
                wait_in(ib)

                @pl.when(t + 1 < gcnt)
                def _():
                    start_in(t + 1, 1 - ib)

                @pl.when(t >= 2)
                def _():
                    wait_out(ib)

                shuffle(ib)
                start_out(t, ib)

        start_in(0, 0)

        def pair(s, carry):
            phase(2 * s, 0)
            phase(2 * s + 1, 1)
            return carry

        lax.fori_loop(0, (_NG_PER_W + 2) // 2 + 1, pair, 0)
        wait_out(0)
        wait_out(1)

        @pl.when(wid == 0)
        def _():
            pltpu.sync_copy(tail_hbm, in_v0.at[pl.ds(0, 16), pl.ds(0, 128)])
            pltpu.sync_copy(in_v0.at[pl.ds(0, 16), pl.ds(0, 128)],
                            sc_hbm.at[pl.ds(249984, 16), :])

    return body(et, tail)


@jax.jit
def _embed_lookup(xflat, table):
    mesh = plsc.VectorSubcoreMesh(core_axis_name="c", subcore_axis_name="s")

    @functools.partial(
        pl.kernel,
        mesh=mesh,
        out_type=jax.ShapeDtypeStruct((20, 32, 16384), jnp.float32),
        scratch_types=[
            pltpu.VMEM((_PER_W,), jnp.int32),
            pltpu.VMEM((_PER_W,), jnp.int32),
            pltpu.VMEM((128, 128), jnp.float32),
            pltpu.VMEM((1, 32, 128), jnp.float32),
            pltpu.SemaphoreType.DMA,
        ],
        compiler_params=pltpu.CompilerParams(needs_layout_passes=False),
    )
    def body(table_hbm, xf_hbm, out_hbm, idx_v, q_v, rows_v, ot_v, sem):
        wid = lax.axis_index("s") * _NC + lax.axis_index("c")
        lo = wid * _PER_W
        pltpu.sync_copy(xf_hbm.at[pl.ds(lo, _PER_W)], idx_v)
        for v in range(_PER_W // 16):
            q_v[pl.ds(v * 16, 16)] = idx_v[pl.ds(v * 16, 16)] >> 2
        lane = lax.iota(jnp.int32, 16)

        def unit(t, carry):
            u = wid * _UNITS + t
            j = u >> 7
            i0 = pl.multiple_of((u & 127) << 7, 128)
            pltpu.async_copy(
                table_hbm.at[q_v.at[pl.ds(t * 128, 128)]], rows_v, sem
            ).wait()
            for lg in range(8):
                r16 = idx_v[pl.ds(t * 128 + lg * 16, 16)]
                row_idx = lane + lg * 16
                col0 = (r16 & 3) << 5
                for e in range(32):
                    ot_v[0, e, pl.ds(lg * 16, 16)] = plsc.load_gather(
                        rows_v, [row_idx, col0 + e])
            pltpu.sync_copy(ot_v, out_hbm.at[pl.ds(j, 1), :, pl.ds(i0, 128)])
            return carry

        lax.fori_loop(0, _UNITS, unit, 0)

    return body(table, xflat)


def kernel(x, embedding):
    if x.dtype != jnp.int32:
        x = jnp.round(x).astype(jnp.int32)
    table = embedding.reshape(250000, 128)
    out_t = _embed_lookup(x.T.reshape(-1), table)
    return jnp.transpose(out_t, (2, 0, 1))


# speedup vs baseline: 1.8338x; 1.1473x over previous
import functools

import jax
import jax.numpy as jnp
from jax import lax
from jax.experimental import pallas as pl
from jax.experimental.pallas import tpu as pltpu, tpu_sc as plsc

_INFO = plsc.get_sparse_core_info()
_NC, _NS = _INFO.num_cores, _INFO.num_subcores
_NW = _NC * _NS

_B = 327680
_PER_W = _B // _NW
_UNITS = _PER_W // 128


_GL = 128
_NG = 999936 // _GL
_NG_PER_W = _NG // _NW


@jax.jit
def _retile(et, tail):
    mesh = plsc.VectorSubcoreMesh(core_axis_name="c", subcore_axis_name="s")

    @functools.partial(
        pl.kernel,
        mesh=mesh,
        out_type=jax.ShapeDtypeStruct((250000, 128), jnp.float32),
        scratch_types=[
            pltpu.VMEM((32, _GL), jnp.float32),
            pltpu.VMEM((32, _GL), jnp.float32),
            pltpu.VMEM((_GL // 4, 128), jnp.float32),
            pltpu.VMEM((_GL // 4, 128), jnp.float32),
            pltpu.SemaphoreType.DMA,
            pltpu.SemaphoreType.DMA,
            pltpu.SemaphoreType.DMA,
            pltpu.SemaphoreType.DMA,
        ],
        compiler_params=pltpu.CompilerParams(needs_layout_passes=False),
    )
    def body(et_hbm, tail_hbm, sc_hbm, in_v0, in_v1, out_v0, out_v1,
             isem0, isem1, osem0, osem1):
        wid = lax.axis_index("s") * _NC + lax.axis_index("c")
        lane = lax.iota(jnp.int32, 16)
        ins = (in_v0, in_v1)
        outs = (out_v0, out_v1)
        isems = (isem0, isem1)
        osems = (osem0, osem1)

        nx = _NG - _NG_PER_W * _NW
        gbase = wid * _NG_PER_W + jnp.minimum(wid, nx)
        gcnt = _NG_PER_W + (wid < nx).astype(jnp.int32)

        def start_in(t, ib):
            c0 = pl.multiple_of((gbase + t) * _GL, 128)
            pltpu.async_copy(et_hbm.at[:, pl.ds(c0, _GL)], ins[ib], isems[ib])

        def wait_in(ib):
            pltpu.make_async_copy(
                et_hbm.at[:, pl.ds(0, _GL)], ins[ib], isems[ib]).wait()

        def start_out(t, ib):
            r0 = pl.multiple_of((gbase + t) * (_GL // 4), 8)
            pltpu.async_copy(
                outs[ib], sc_hbm.at[pl.ds(r0, _GL // 4), :], osems[ib])

        def wait_out(ib):
            pltpu.make_async_copy(
                outs[ib], sc_hbm.at[pl.ds(0, _GL // 4), :], osems[ib]).wait()

        def shuffle(ib):
            for qq in range(_GL // 4):
                for lg in range(8):
                    row_idx = ((lg & 1) << 4) + lane
                    colv = lane * 0 + (4 * qq + (lg >> 1))
                    outs[ib][qq, pl.ds(lg * 16, 16)] = plsc.load_gather(
                        ins[ib], [row_idx, colv])

        def phase(t, ib):
            @pl.when(t < gcnt)
            def _():
                wait_in(ib)

                @pl.when(t + 1 < gcnt)
                def _():
                    start_in(t + 1, 1 - ib)

                @pl.when(t >= 2)
                def _():
                    wait_out(ib)

                shuffle(ib)
                start_out(t, ib)

        start_in(0, 0)

        def pair(s, carry):
            phase(2 * s, 0)
            phase(2 * s + 1, 1)
            return carry

        lax.fori_loop(0, (_NG_PER_W + 2) // 2 + 1, pair, 0)
        wait_out(0)
        wait_out(1)

        @pl.when(wid == 0)
        def _():
            pltpu.sync_copy(tail_hbm, in_v0.at[pl.ds(0, 16), pl.ds(0, 128)])
            pltpu.sync_copy(in_v0.at[pl.ds(0, 16), pl.ds(0, 128)],
                            sc_hbm.at[pl.ds(249984, 16), :])

    return body(et, tail)


@jax.jit
def _embed_lookup(xflat, table):
    mesh = plsc.VectorSubcoreMesh(core_axis_name="c", subcore_axis_name="s")

    @functools.partial(
        pl.kernel,
        mesh=mesh,
        out_type=jax.ShapeDtypeStruct((20, 32, 16384), jnp.float32),
        scratch_types=[
            pltpu.VMEM((_PER_W,), jnp.int32),
            pltpu.VMEM((_PER_W,), jnp.int32),
            pltpu.VMEM((128, 128), jnp.float32),
            pltpu.VMEM((128, 128), jnp.float32),
            pltpu.VMEM((1, 32, 128), jnp.float32),
            pltpu.VMEM((1, 32, 128), jnp.float32),
            pltpu.SemaphoreType.DMA,
            pltpu.SemaphoreType.DMA,
            pltpu.SemaphoreType.DMA,
            pltpu.SemaphoreType.DMA,
        ],
        compiler_params=pltpu.CompilerParams(needs_layout_passes=False),
    )
    def body(table_hbm, xf_hbm, out_hbm, idx_v, q_v, rows_v0, rows_v1,
             ot_v0, ot_v1, gsem0, gsem1, osem0, osem1):
        wid = lax.axis_index("s") * _NC + lax.axis_index("c")
        lo = wid * _PER_W
        rows = (rows_v0, rows_v1)
        ots = (ot_v0, ot_v1)
        gsems = (gsem0, gsem1)
        osems = (osem0, osem1)
        pltpu.sync_copy(xf_hbm.at[pl.ds(lo, _PER_W)], idx_v)
        for v in range(_PER_W // 16):
            q_v[pl.ds(v * 16, 16)] = idx_v[pl.ds(v * 16, 16)] >> 2
        lane = lax.iota(jnp.int32, 16)

        def start_gather(t, ib):
            pltpu.async_copy(
                table_hbm.at[q_v.at[pl.ds(t * 128, 128)]], rows[ib], gsems[ib])

        def wait_gather(ib):
            pltpu.make_async_copy(
                table_hbm.at[q_v.at[pl.ds(0, 128)]], rows[ib],
                gsems[ib]).wait()

        def start_out(t, ib):
            u = wid * _UNITS + t
            j = u >> 7
            i0 = pl.multiple_of((u & 127) << 7, 128)
            pltpu.async_copy(
                ots[ib], out_hbm.at[pl.ds(j, 1), :, pl.ds(i0, 128)],
                osems[ib])

        def wait_out(ib):
            pltpu.make_async_copy(
                ots[ib], out_hbm.at[pl.ds(0, 1), :, pl.ds(0, 128)],
                osems[ib]).wait()

        def phase(s, t, ib):
            wait_gather(ib)

            @pl.when(t + 1 < _UNITS)
            def _():
                start_gather(t + 1, 1 - ib)

            @pl.when(s > 0)
            def _():
                wait_out(ib)

            for lg in range(8):
                r16 = idx_v[pl.ds(t * 128 + lg * 16, 16)]
                row_idx = lane + lg * 16
                col0 = (r16 & 3) << 5
                for e in range(32):
                    ots[ib][0, e, pl.ds(lg * 16, 16)] = plsc.load_gather(
                        rows[ib], [row_idx, col0 + e])
            start_out(t, ib)

        start_gather(0, 0)

        def pair(s, carry):
            phase(s, 2 * s, 0)
            phase(s, 2 * s + 1, 1)
            return carry

        lax.fori_loop(0, _UNITS // 2, pair, 0)
        wait_out(0)
        wait_out(1)

    return body(table, xflat)


def kernel(x, embedding):
    if x.dtype != jnp.int32:
        x = jnp.round(x).astype(jnp.int32)
    table = embedding.reshape(250000, 128)
    out_t = _embed_lookup(x.T.reshape(-1), table)
    return jnp.transpose(out_t, (2, 0, 1))


# speedup vs baseline: 1.9557x; 1.0665x over previous
import functools

import jax
import jax.numpy as jnp
from jax import lax
from jax.experimental import pallas as pl
from jax.experimental.pallas import tpu as pltpu, tpu_sc as plsc

_INFO = plsc.get_sparse_core_info()
_NC, _NS = _INFO.num_cores, _INFO.num_subcores
_NW = _NC * _NS

_CHUNK = 640
_NBUF = 4


@functools.partial(jax.jit, static_argnames=("n_rows", "n_cols"))
def _embed_lookup(x_flat, embedding, *, n_rows, n_cols):
    b = x_flat.shape[0]
    b_per_w = b // _NW
    n_chunks = b_per_w // _CHUNK
    d = embedding.shape[1]

    mesh = plsc.VectorSubcoreMesh(core_axis_name="c", subcore_axis_name="s")

    @functools.partial(
        pl.kernel,
        mesh=mesh,
        out_type=jax.ShapeDtypeStruct((b, d), jnp.float32),
        scratch_types=(
            [pltpu.VMEM((b_per_w,), jnp.int32)]
            + [pltpu.VMEM((_CHUNK, d), jnp.float32) for _ in range(_NBUF)]
            + [pltpu.SemaphoreType.DMA for _ in range(2 * _NBUF)]
        ),
        compiler_params=pltpu.CompilerParams(use_tc_tiling_on_sc=False),
    )
    def body(table_hbm, idx_hbm, out_hbm, idx_v, *bufs_and_sems):
        rows = bufs_and_sems[:_NBUF]
        gsems = bufs_and_sems[_NBUF:2 * _NBUF]
        wsems = bufs_and_sems[2 * _NBUF:]
        wid = lax.axis_index("s") * _NC + lax.axis_index("c")
        base = wid * b_per_w
        pltpu.sync_copy(idx_hbm.at[pl.ds(base, b_per_w)], idx_v)
        gathers = [None] * _NBUF
        writes = [None] * _NBUF
        for c in range(n_chunks + _NBUF - 1):
            if c < n_chunks:
                ib = c % _NBUF
                if writes[ib] is not None:
                    writes[ib].wait()
                gathers[ib] = pltpu.async_copy(
                    table_hbm.at[idx_v.at[pl.ds(c * _CHUNK, _CHUNK)]],
                    rows[ib], gsems[ib])
            cd = c - (_NBUF - 1)
            if cd >= 0:
                db = cd % _NBUF
                gathers[db].wait()
                writes[db] = pltpu.async_copy(
                    rows[db],
                    out_hbm.at[pl.ds(base + cd * _CHUNK, _CHUNK)],
                    wsems[db])
        for w in writes:
            w.wait()

    out = body(embedding, x_flat)
    return out.reshape(n_rows, n_cols, d)


def kernel(x, embedding):
    if x.dtype != jnp.int32:
        x = jnp.round(x).astype(jnp.int32)
    n_rows, n_cols = x.shape
    return _embed_lookup(x.reshape(-1), embedding, n_rows=n_rows, n_cols=n_cols)
